# baseline (speedup 1.0000x reference)
import functools
import math

import jax
import jax.numpy as jnp
from jax import lax
from jax.experimental import pallas as pl
from jax.experimental.pallas import tpu as pltpu
from jax.experimental.pallas import tpu_sc as plsc

B, H, T, DH, D = 2, 16, 4096, 64, 64
NEG = -1e30
NW = 32
RPT = (H * T) // NW
SQ = 64
NSC = RPT // SQ
CH = 1
NCH = SQ // CH


def _attn_kernel(kv_hbm, qf_hbm, idx_hbm, et_hbm, bt_hbm, out_hbm,
                 bias_v, q_v, idx_v, et_v, comb_v, adj_v, out_v,
                 rows0, rows1, rows2, rows3, sem0, sem1, sem2, sem3):
    wid = lax.axis_index("s") * 2 + lax.axis_index("c")
    h = wid // 2
    t0 = (wid % 2) * RPT
    row0 = wid * RPT
    hbase = h * T

    pltpu.sync_copy(bt_hbm, bias_v)

    iota16 = lax.iota(jnp.int32, 16)

    perm1 = (iota16 + 8) & 15
    perm2 = (iota16 & 8) | ((iota16 + 4) & 7)
    perm3 = (iota16 & 12) | ((iota16 + 2) & 3)
    perm4 = iota16 ^ 1
    m3 = (iota16 & 8) == 0
    m2 = (iota16 & 4) == 0
    m1 = (iota16 & 2) == 0
    m0 = (iota16 & 1) == 0
    brev = ((iota16 & 1) << 3) | ((iota16 & 2) << 1) \
        | ((iota16 & 4) >> 1) | ((iota16 & 8) >> 3)
    splats = [jnp.full((16,), u, jnp.int32) for u in range(16)]

    def rgather(x, p):
        return x.at[p].get(mode="promise_in_bounds")

    def butterfly(a):
        s = [a[j] + rgather(a[j], perm1) for j in range(16)]
        c = [jnp.where(m3, s[2 * i], s[2 * i + 1]) for i in range(8)]
        t = [c[j] + rgather(c[j], perm2) for j in range(8)]
        d = [jnp.where(m2, t[2 * i], t[2 * i + 1]) for i in range(4)]
        u = [d[j] + rgather(d[j], perm3) for j in range(4)]
        e = [jnp.where(m1, u[2 * i], u[2 * i + 1]) for i in range(2)]
        v = [e[j] + rgather(e[j], perm4) for j in range(2)]
        r = jnp.where(m0, v[0], v[1])
        return rgather(r, brev)

    def issue(g, buf, sem):
        pltpu.async_copy(kv_hbm.at[adj_v.at[g]], buf, sem)

    def wait(g, buf, sem):
        pltpu.make_async_copy(kv_hbm.at[adj_v.at[g]], buf, sem).wait()

    def compute_chunk(g, rows):
        qrow = g
        qv = [[q_v[qrow, pl.ds(b * DH + j * 16, 16)] for j in range(4)]
              for b in range(2)]

        def unpk(r, woff, j):
            x = plsc.bitcast(rows[r, pl.ds(woff + j * 16, 16)], jnp.bfloat16)
            return plsc.unpack(x, format=plsc.PackFormat.INTERLEAVED,
                               preferred_element_type=jnp.float32)

        ws = [[], []]
        for b in range(2):
            koff = b * DH
            masked = []
            for grp in range(4):
                accs = []
                for u in range(16):
                    r = grp * 16 + u
                    ev0, od0 = unpk(r, koff, 0)
                    a = ev0 * qv[b][0] + od0 * qv[b][1]
                    ev1, od1 = unpk(r, koff, 1)
                    a = a + ev1 * qv[b][2] + od1 * qv[b][3]
                    accs.append(a)
                sc = butterfly(accs) * (1.0 / math.sqrt(DH))
                masked.append(sc + comb_v[qrow, pl.ds(grp * 16, 16)])
            m = jnp.maximum(jnp.maximum(masked[0], masked[1]),
                            jnp.maximum(masked[2], masked[3]))
            mx = jnp.max(m)
            es = [jnp.where(mm > -1e29, jnp.exp(mm - mx),
                            jnp.zeros((16,), jnp.float32))
                  for mm in masked]
            dn = jnp.sum(es[0] + es[1] + es[2] + es[3])
            dnv = jnp.maximum(jnp.full((16,), dn, jnp.float32), 1e-9)
            inv = jnp.full((16,), 1.0, jnp.float32) / dnv
            ws[b] = [e * inv for e in es]

        ys = [jnp.zeros((16,), jnp.float32) for _ in range(8)]
        for d4 in range(4):
            for u in range(16):
                r = d4 * 16 + u
                w0 = rgather(ws[0][d4], splats[u])
                w1 = rgather(ws[1][d4], splats[u])
                for b, w in ((0, w0), (1, w1)):
                    voff = 32 + b * DH
                    for j in range(2):
                        ev, od = unpk(r, voff, j)
                        ys[b * 4 + 2 * j] = ys[b * 4 + 2 * j] + ev * w
                        ys[b * 4 + 2 * j + 1] = ys[b * 4 + 2 * j + 1] + od * w
        for mreg in range(4):
            out_v[qrow, pl.ds(mreg * 16, 16)] = ys[mreg]
            out_v[qrow, pl.ds(DH + mreg * 16, 16)] = ys[4 + mreg]

    def sc_body(s, _):
        base = row0 + s * SQ
        t_base = t0 + s * SQ
        pltpu.sync_copy(qf_hbm.at[pl.ds(base, SQ)], q_v)
        pltpu.sync_copy(idx_hbm.at[pl.ds(base, SQ)], idx_v)
        pltpu.sync_copy(et_hbm.at[pl.ds(base, SQ)], et_v)

        def pre_body(g, _):
            for m in range(4):
                qrow = g
                coff = m * 16
                raw = idx_v[qrow, pl.ds(coff, 16)]
                et16 = et_v[qrow, pl.ds(coff, 16)]
                b16 = plsc.load_gather(bias_v, [et16])
                msk = raw <= (t_base + qrow)
                comb_v[qrow, pl.ds(coff, 16)] = jnp.where(
                    msk, b16, jnp.full((16,), NEG, jnp.float32))
                adj_v[g, pl.ds(coff, 16)] = raw + hbase
            return 0

        lax.fori_loop(0, NCH, pre_body, 0)

        issue(0, rows0, sem0)
        issue(1, rows1, sem1)
        issue(2, rows2, sem2)

        def ch_body(i, _):
            g0 = 4 * i
            issue(g0 + 3, rows3, sem3)
            wait(g0, rows0, sem0)
            compute_chunk(g0, rows0)

            @pl.when(i < NCH // 4 - 1)
            def _():
                issue(g0 + 4, rows0, sem0)

            wait(g0 + 1, rows1, sem1)
            compute_chunk(g0 + 1, rows1)

            @pl.when(i < NCH // 4 - 1)
            def _():
                issue(g0 + 5, rows1, sem1)

            wait(g0 + 2, rows2, sem2)
            compute_chunk(g0 + 2, rows2)

            @pl.when(i < NCH // 4 - 1)
            def _():
                issue(g0 + 6, rows2, sem2)

            wait(g0 + 3, rows3, sem3)
            compute_chunk(g0 + 3, rows3)
            return 0

        lax.fori_loop(0, NCH // 4, ch_body, 0)
        pltpu.sync_copy(out_v, out_hbm.at[pl.ds(base, SQ)])
        return 0

    lax.fori_loop(0, NSC, sc_body, 0)


@functools.partial(
    pl.kernel,
    out_type=jax.ShapeDtypeStruct((H * T, 2 * DH), jnp.float32),
    mesh=plsc.VectorSubcoreMesh(core_axis_name="c", subcore_axis_name="s"),
    compiler_params=pltpu.CompilerParams(needs_layout_passes=False),
    scratch_types=[
        pltpu.VMEM((16,), jnp.float32),
        pltpu.VMEM((SQ, 2 * DH), jnp.float32),
        pltpu.VMEM((SQ, D), jnp.int32),
        pltpu.VMEM((SQ, D), jnp.int32),
        pltpu.VMEM((SQ, D), jnp.float32),
        pltpu.VMEM((NCH, CH * D), jnp.int32),
        pltpu.VMEM((SQ, 2 * DH), jnp.float32),
        pltpu.VMEM((CH * D, 2 * DH), jnp.int32),
        pltpu.VMEM((CH * D, 2 * DH), jnp.int32),
        pltpu.VMEM((CH * D, 2 * DH), jnp.int32),
        pltpu.VMEM((CH * D, 2 * DH), jnp.int32),
        pltpu.SemaphoreType.DMA,
        pltpu.SemaphoreType.DMA,
        pltpu.SemaphoreType.DMA,
        pltpu.SemaphoreType.DMA,
    ],
)
def _sc_attention(kv_hbm, qf_hbm, idx_hbm, et_hbm, bt_hbm, out_hbm,
                  *scratch):
    _attn_kernel(kv_hbm, qf_hbm, idx_hbm, et_hbm, bt_hbm, out_hbm, *scratch)


@jax.jit
def kernel(q, k, v, neigh_idx, edge_type, edge_type_bias):
    kvf = jnp.concatenate([k[0], v[0], k[1], v[1]], axis=-1)
    kvf = kvf.reshape(H * T, 4 * DH).astype(jnp.bfloat16)
    kvp = jax.lax.bitcast_convert_type(
        kvf.reshape(H * T, 2 * DH, 2), jnp.int32)
    qf = jnp.concatenate([q[0], q[1]], axis=-1).reshape(H * T, 2 * DH)
    qd = qf.reshape(H * T, 2, 2, 16, 2).transpose(0, 1, 2, 4, 3)
    qd = qd.reshape(H * T, 2 * DH).astype(jnp.float32)
    idx32 = neigh_idx.astype(jnp.int32).reshape(H * T, D)
    et32 = edge_type.astype(jnp.int32).reshape(H * T, D)
    btab = jnp.zeros((16,), jnp.float32)
    btab = btab.at[1:5].set(edge_type_bias.astype(jnp.float32))
    out = _sc_attention(kvp, qd, idx32, et32, btab)
    y = out.reshape(H, T, 2, 2, 2, 16).transpose(2, 0, 1, 3, 5, 4)
    y = y.reshape(2, H, T, DH)
    return y.astype(v.dtype)


# speedup vs baseline: 2.1736x; 2.1736x over previous
import functools
import math

import jax
import jax.numpy as jnp
from jax import lax
from jax.experimental import pallas as pl
from jax.experimental.pallas import tpu as pltpu
from jax.experimental.pallas import tpu_sc as plsc

B, H, T, DH, D = 2, 16, 4096, 64, 64
NEG = -1e30
NW = 32
RPT = (H * T) // NW
SQ = 64
NSC = RPT // SQ
CH = 1
NCH = SQ // CH


def _attn_kernel(kv_hbm, qf_hbm, idx_hbm, et_hbm, bt_hbm, out_hbm,
                 bias_v, q_v, idx_v, et_v, comb_v, adj_v, out_v,
                 rows_v, sems):
    wid = lax.axis_index("s") * 2 + lax.axis_index("c")
    h = wid // 2
    t0 = (wid % 2) * RPT
    row0 = wid * RPT
    hbase = h * T

    pltpu.sync_copy(bt_hbm, bias_v)

    iota16 = lax.iota(jnp.int32, 16)

    perm1 = (iota16 + 8) & 15
    perm2 = (iota16 & 8) | ((iota16 + 4) & 7)
    perm3 = (iota16 & 12) | ((iota16 + 2) & 3)
    perm4 = iota16 ^ 1
    m3 = (iota16 & 8) == 0
    m2 = (iota16 & 4) == 0
    m1 = (iota16 & 2) == 0
    m0 = (iota16 & 1) == 0
    brev = ((iota16 & 1) << 3) | ((iota16 & 2) << 1) \
        | ((iota16 & 4) >> 1) | ((iota16 & 8) >> 3)
    splats = [jnp.full((16,), u, jnp.int32) for u in range(16)]

    def rgather(x, p):
        return x.at[p].get(mode="promise_in_bounds")

    def butterfly(a):
        s = [a[j] + rgather(a[j], perm1) for j in range(16)]
        c = [jnp.where(m3, s[2 * i], s[2 * i + 1]) for i in range(8)]
        t = [c[j] + rgather(c[j], perm2) for j in range(8)]
        d = [jnp.where(m2, t[2 * i], t[2 * i + 1]) for i in range(4)]
        u = [d[j] + rgather(d[j], perm3) for j in range(4)]
        e = [jnp.where(m1, u[2 * i], u[2 * i + 1]) for i in range(2)]
        v = [e[j] + rgather(e[j], perm4) for j in range(2)]
        r = jnp.where(m0, v[0], v[1])
        return rgather(r, brev)

    def issue(g, par):
        pltpu.async_copy(kv_hbm.at[adj_v.at[g]], rows_v.at[par],
                         sems.at[par])

    def wait(g, par):
        pltpu.make_async_copy(kv_hbm.at[adj_v.at[g]], rows_v.at[par],
                              sems.at[par]).wait()

    def compute_chunk(g, par):
        qrow = g
        qv = [[q_v[qrow, pl.ds(b * DH + j * 16, 16)] for j in range(4)]
              for b in range(2)]

        def unpk(r, woff, j):
            x = plsc.bitcast(rows_v[par, r, pl.ds(woff + j * 16, 16)],
                             jnp.bfloat16)
            return plsc.unpack(x, format=plsc.PackFormat.INTERLEAVED,
                               preferred_element_type=jnp.float32)

        ws = [[], []]
        for b in range(2):
            koff = b * DH
            masked = []
            for grp in range(4):
                accs = []
                for u in range(16):
                    r = grp * 16 + u
                    ev0, od0 = unpk(r, koff, 0)
                    a = ev0 * qv[b][0] + od0 * qv[b][1]
                    ev1, od1 = unpk(r, koff, 1)
                    a = a + ev1 * qv[b][2] + od1 * qv[b][3]
                    accs.append(a)
                sc = butterfly(accs) * (1.0 / math.sqrt(DH))
                masked.append(sc + comb_v[qrow, pl.ds(grp * 16, 16)])
            m = jnp.maximum(jnp.maximum(masked[0], masked[1]),
                            jnp.maximum(masked[2], masked[3]))
            mx = jnp.max(m)
            es = [jnp.where(mm > -1e29, jnp.exp(mm - mx),
                            jnp.zeros((16,), jnp.float32))
                  for mm in masked]
            dn = jnp.sum(es[0] + es[1] + es[2] + es[3])
            dnv = jnp.maximum(jnp.full((16,), dn, jnp.float32), 1e-9)
            inv = jnp.full((16,), 1.0, jnp.float32) / dnv
            ws[b] = [e * inv for e in es]

        ys = [jnp.zeros((16,), jnp.float32) for _ in range(8)]
        for d4 in range(4):
            for u in range(16):
                r = d4 * 16 + u
                w0 = rgather(ws[0][d4], splats[u])
                w1 = rgather(ws[1][d4], splats[u])
                for b, w in ((0, w0), (1, w1)):
                    voff = 32 + b * DH
                    for j in range(2):
                        ev, od = unpk(r, voff, j)
                        ys[b * 4 + 2 * j] = ys[b * 4 + 2 * j] + ev * w
                        ys[b * 4 + 2 * j + 1] = ys[b * 4 + 2 * j + 1] + od * w
        for mreg in range(4):
            out_v[qrow, pl.ds(mreg * 16, 16)] = ys[mreg]
            out_v[qrow, pl.ds(DH + mreg * 16, 16)] = ys[4 + mreg]

    def sc_body(s, _):
        base = row0 + s * SQ
        t_base = t0 + s * SQ
        pltpu.sync_copy(qf_hbm.at[pl.ds(base, SQ)], q_v)
        pltpu.sync_copy(idx_hbm.at[pl.ds(base, SQ)], idx_v)
        pltpu.sync_copy(et_hbm.at[pl.ds(base, SQ)], et_v)

        def pre_body(g, _):
            for m in range(4):
                qrow = g
                coff = m * 16
                raw = idx_v[qrow, pl.ds(coff, 16)]
                et16 = et_v[qrow, pl.ds(coff, 16)]
                b16 = plsc.load_gather(bias_v, [et16])
                msk = raw <= (t_base + qrow)
                comb_v[qrow, pl.ds(coff, 16)] = jnp.where(
                    msk, b16, jnp.full((16,), NEG, jnp.float32))
                adj_v[g, pl.ds(coff, 16)] = raw + hbase
            return 0

        lax.fori_loop(0, NCH, pre_body, 0)

        issue(0, 0)
        issue(1, 1)

        def ch_body(g, _):
            par = g & 1
            wait(g, par)
            compute_chunk(g, par)

            @pl.when(g < NCH - 2)
            def _():
                issue(g + 2, par)

            return 0

        lax.fori_loop(0, NCH, ch_body, 0)
        pltpu.sync_copy(out_v, out_hbm.at[pl.ds(base, SQ)])
        return 0

    lax.fori_loop(0, NSC, sc_body, 0)


@functools.partial(
    pl.kernel,
    out_type=jax.ShapeDtypeStruct((H * T, 2 * DH), jnp.float32),
    mesh=plsc.VectorSubcoreMesh(core_axis_name="c", subcore_axis_name="s"),
    compiler_params=pltpu.CompilerParams(needs_layout_passes=False),
    scratch_types=[
        pltpu.VMEM((16,), jnp.float32),
        pltpu.VMEM((SQ, 2 * DH), jnp.float32),
        pltpu.VMEM((SQ, D), jnp.int32),
        pltpu.VMEM((SQ, D), jnp.int32),
        pltpu.VMEM((SQ, D), jnp.float32),
        pltpu.VMEM((NCH, CH * D), jnp.int32),
        pltpu.VMEM((SQ, 2 * DH), jnp.float32),
        pltpu.VMEM((2, CH * D, 2 * DH), jnp.int32),
        pltpu.SemaphoreType.DMA((2,)),
    ],
)
def _sc_attention(kv_hbm, qf_hbm, idx_hbm, et_hbm, bt_hbm, out_hbm,
                  *scratch):
    _attn_kernel(kv_hbm, qf_hbm, idx_hbm, et_hbm, bt_hbm, out_hbm, *scratch)


@jax.jit
def kernel(q, k, v, neigh_idx, edge_type, edge_type_bias):
    kvf = jnp.concatenate([k[0], v[0], k[1], v[1]], axis=-1)
    kvf = kvf.reshape(H * T, 4 * DH).astype(jnp.bfloat16)
    kvp = jax.lax.bitcast_convert_type(
        kvf.reshape(H * T, 2 * DH, 2), jnp.int32)
    qf = jnp.concatenate([q[0], q[1]], axis=-1).reshape(H * T, 2 * DH)
    qd = qf.reshape(H * T, 2, 2, 16, 2).transpose(0, 1, 2, 4, 3)
    qd = qd.reshape(H * T, 2 * DH).astype(jnp.float32)
    idx32 = neigh_idx.astype(jnp.int32).reshape(H * T, D)
    et32 = edge_type.astype(jnp.int32).reshape(H * T, D)
    btab = jnp.zeros((16,), jnp.float32)
    btab = btab.at[1:5].set(edge_type_bias.astype(jnp.float32))
    out = _sc_attention(kvp, qd, idx32, et32, btab)
    y = out.reshape(H, T, 2, 2, 2, 16).transpose(2, 0, 1, 3, 5, 4)
    y = y.reshape(2, H, T, DH)
    return y.astype(v.dtype)


# speedup vs baseline: 2.3928x; 1.1008x over previous
import functools
import math

import jax
import jax.numpy as jnp
from jax import lax
from jax.experimental import pallas as pl
from jax.experimental.pallas import tpu as pltpu
from jax.experimental.pallas import tpu_sc as plsc

B, H, T, DH, D = 2, 16, 4096, 64, 64
NEG = -1e30
NW = 32
RPT = (H * T) // NW
SQ = 64
NSC = RPT // SQ
CH = 1
NCH = SQ // CH


def _attn_kernel(kv_hbm, qf_hbm, idx_hbm, et_hbm, bt_hbm, out_hbm,
                 bias_v, q_v, idx_v, et_v, comb_v, adj_v, out_v,
                 rows_v, sems):
    wid = lax.axis_index("s") * 2 + lax.axis_index("c")
    h = wid // 2
    t0 = (wid % 2) * RPT
    row0 = wid * RPT
    hbase = h * T

    pltpu.sync_copy(bt_hbm, bias_v)

    iota16 = lax.iota(jnp.int32, 16)

    perm1 = (iota16 + 8) & 15
    perm2 = (iota16 & 8) | ((iota16 + 4) & 7)
    perm3 = (iota16 & 12) | ((iota16 + 2) & 3)
    perm4 = iota16 ^ 1
    m3 = (iota16 & 8) == 0
    m2 = (iota16 & 4) == 0
    m1 = (iota16 & 2) == 0
    m0 = (iota16 & 1) == 0
    brev = ((iota16 & 1) << 3) | ((iota16 & 2) << 1) \
        | ((iota16 & 4) >> 1) | ((iota16 & 8) >> 3)
    splats = [jnp.full((16,), u, jnp.int32) for u in range(16)]

    def rgather(x, p):
        return x.at[p].get(mode="promise_in_bounds")

    def butterfly(a):
        s = [a[j] + rgather(a[j], perm1) for j in range(16)]
        c = [jnp.where(m3, s[2 * i], s[2 * i + 1]) for i in range(8)]
        t = [c[j] + rgather(c[j], perm2) for j in range(8)]
        d = [jnp.where(m2, t[2 * i], t[2 * i + 1]) for i in range(4)]
        u = [d[j] + rgather(d[j], perm3) for j in range(4)]
        e = [jnp.where(m1, u[2 * i], u[2 * i + 1]) for i in range(2)]
        v = [e[j] + rgather(e[j], perm4) for j in range(2)]
        r = jnp.where(m0, v[0], v[1])
        return rgather(r, brev)

    def issue(g, par):
        pltpu.async_copy(kv_hbm.at[adj_v.at[g]], rows_v.at[par],
                         sems.at[par])

    def wait(g, par):
        pltpu.make_async_copy(kv_hbm.at[adj_v.at[g]], rows_v.at[par],
                              sems.at[par]).wait()

    def compute_chunk(g, par):
        qrow = g
        qv = [[q_v[qrow, pl.ds(b * DH + j * 16, 16)] for j in range(4)]
              for b in range(2)]

        def unpk(r, woff, j):
            x = plsc.bitcast(rows_v[par, r, pl.ds(woff + j * 16, 16)],
                             jnp.bfloat16)
            return plsc.unpack(x, format=plsc.PackFormat.INTERLEAVED,
                               preferred_element_type=jnp.float32)

        ws = [[], []]
        for b in range(2):
            koff = b * DH
            masked = []
            for grp in range(4):
                accs = []
                for u in range(16):
                    r = grp * 16 + u
                    ev0, od0 = unpk(r, koff, 0)
                    a = ev0 * qv[b][0] + od0 * qv[b][1]
                    ev1, od1 = unpk(r, koff, 1)
                    a = a + ev1 * qv[b][2] + od1 * qv[b][3]
                    accs.append(a)
                sc = butterfly(accs) * (1.0 / math.sqrt(DH))
                masked.append(sc + comb_v[qrow, pl.ds(grp * 16, 16)])
            m = jnp.maximum(jnp.maximum(masked[0], masked[1]),
                            jnp.maximum(masked[2], masked[3]))
            mx = jnp.max(m)
            es = [jnp.where(mm > -1e29, jnp.exp(mm - mx),
                            jnp.zeros((16,), jnp.float32))
                  for mm in masked]
            dn = jnp.sum(es[0] + es[1] + es[2] + es[3])
            dnv = jnp.maximum(jnp.full((16,), dn, jnp.float32), 1e-9)
            inv = jnp.full((16,), 1.0, jnp.float32) / dnv
            ws[b] = [e * inv for e in es]

        ys = [jnp.zeros((16,), jnp.float32) for _ in range(8)]
        for d4 in range(4):
            for u in range(16):
                r = d4 * 16 + u
                w0 = rgather(ws[0][d4], splats[u])
                w1 = rgather(ws[1][d4], splats[u])
                for b, w in ((0, w0), (1, w1)):
                    voff = 32 + b * DH
                    for j in range(2):
                        ev, od = unpk(r, voff, j)
                        ys[b * 4 + 2 * j] = ys[b * 4 + 2 * j] + ev * w
                        ys[b * 4 + 2 * j + 1] = ys[b * 4 + 2 * j + 1] + od * w
        for mreg in range(4):
            out_v[qrow, pl.ds(mreg * 16, 16)] = ys[mreg]
            out_v[qrow, pl.ds(DH + mreg * 16, 16)] = ys[4 + mreg]

    def sc_body(s, _):
        base = row0 + s * SQ
        t_base = t0 + s * SQ
        pltpu.sync_copy(qf_hbm.at[pl.ds(base, SQ)], q_v)
        pltpu.sync_copy(idx_hbm.at[pl.ds(base, SQ)], idx_v)
        pltpu.sync_copy(et_hbm.at[pl.ds(base, SQ)], et_v)

        def pre_body(g, _):
            for m in range(4):
                qrow = g
                coff = m * 16
                raw = idx_v[qrow, pl.ds(coff, 16)]
                et16 = et_v[qrow, pl.ds(coff, 16)]
                b16 = plsc.load_gather(bias_v, [et16])
                msk = raw <= (t_base + qrow)
                comb_v[qrow, pl.ds(coff, 16)] = jnp.where(
                    msk, b16, jnp.full((16,), NEG, jnp.float32))
                adj_v[g, pl.ds(coff, 16)] = raw + hbase
            return 0

        lax.fori_loop(0, NCH, pre_body, 0)

        issue(0, 0)
        issue(1, 1)
        issue(2, 2)
        issue(3, 3)

        def ch_body(g, _):
            par = g & 3
            wait(g, par)
            compute_chunk(g, par)

            @pl.when(g < NCH - 4)
            def _():
                issue(g + 4, par)

            return 0

        lax.fori_loop(0, NCH, ch_body, 0)
        pltpu.sync_copy(out_v, out_hbm.at[pl.ds(base, SQ)])
        return 0

    lax.fori_loop(0, NSC, sc_body, 0)


@functools.partial(
    pl.kernel,
    out_type=jax.ShapeDtypeStruct((H * T, 2 * DH), jnp.float32),
    mesh=plsc.VectorSubcoreMesh(core_axis_name="c", subcore_axis_name="s"),
    compiler_params=pltpu.CompilerParams(needs_layout_passes=False),
    scratch_types=[
        pltpu.VMEM((16,), jnp.float32),
        pltpu.VMEM((SQ, 2 * DH), jnp.float32),
        pltpu.VMEM((SQ, D), jnp.int32),
        pltpu.VMEM((SQ, D), jnp.int32),
        pltpu.VMEM((SQ, D), jnp.float32),
        pltpu.VMEM((NCH, CH * D), jnp.int32),
        pltpu.VMEM((SQ, 2 * DH), jnp.float32),
        pltpu.VMEM((4, CH * D, 2 * DH), jnp.int32),
        pltpu.SemaphoreType.DMA((4,)),
    ],
)
def _sc_attention(kv_hbm, qf_hbm, idx_hbm, et_hbm, bt_hbm, out_hbm,
                  *scratch):
    _attn_kernel(kv_hbm, qf_hbm, idx_hbm, et_hbm, bt_hbm, out_hbm, *scratch)


@jax.jit
def kernel(q, k, v, neigh_idx, edge_type, edge_type_bias):
    kvf = jnp.concatenate([k[0], v[0], k[1], v[1]], axis=-1)
    kvf = kvf.reshape(H * T, 4 * DH).astype(jnp.bfloat16)
    kvp = jax.lax.bitcast_convert_type(
        kvf.reshape(H * T, 2 * DH, 2), jnp.int32)
    qf = jnp.concatenate([q[0], q[1]], axis=-1).reshape(H * T, 2 * DH)
    qd = qf.reshape(H * T, 2, 2, 16, 2).transpose(0, 1, 2, 4, 3)
    qd = qd.reshape(H * T, 2 * DH).astype(jnp.float32)
    idx32 = neigh_idx.astype(jnp.int32).reshape(H * T, D)
    et32 = edge_type.astype(jnp.int32).reshape(H * T, D)
    btab = jnp.zeros((16,), jnp.float32)
    btab = btab.at[1:5].set(edge_type_bias.astype(jnp.float32))
    out = _sc_attention(kvp, qd, idx32, et32, btab)
    y = out.reshape(H, T, 2, 2, 2, 16).transpose(2, 0, 1, 3, 5, 4)
    y = y.reshape(2, H, T, DH)
    return y.astype(v.dtype)


# speedup vs baseline: 8.4419x; 3.5281x over previous
import functools
import math

import jax
import jax.numpy as jnp
from jax import lax
from jax.experimental import pallas as pl
from jax.experimental.pallas import tpu as pltpu
from jax.experimental.pallas import tpu_sc as plsc

B, H, T, DH, D = 2, 16, 4096, 64, 64
NEG = -1e30
NW = 32
RPT = (H * T) // NW
SQ = 64
NSC = RPT // SQ
CH = 2
NCH = SQ // CH


def _attn_kernel(kv_hbm, qf_hbm, idx_hbm, et_hbm, bt_hbm, out_hbm,
                 bias_v, q_v, idx_v, et_v, comb_v, adj_v, out_v,
                 rows_v, sems):
    wid = lax.axis_index("s") * 2 + lax.axis_index("c")
    h = wid // 2
    t0 = (wid % 2) * RPT
    row0 = wid * RPT
    hbase = h * T

    pltpu.sync_copy(bt_hbm, bias_v)

    iota16 = lax.iota(jnp.int32, 16)

    perm1 = (iota16 + 8) & 15
    perm2 = (iota16 & 8) | ((iota16 + 4) & 7)
    perm3 = (iota16 & 12) | ((iota16 + 2) & 3)
    perm4 = iota16 ^ 1
    m3 = (iota16 & 8) == 0
    m2 = (iota16 & 4) == 0
    m1 = (iota16 & 2) == 0
    m0 = (iota16 & 1) == 0
    brev = ((iota16 & 1) << 3) | ((iota16 & 2) << 1) \
        | ((iota16 & 4) >> 1) | ((iota16 & 8) >> 3)
    splats = [jnp.full((16,), u, jnp.int32) for u in range(16)]

    def rgather(x, p):
        return x.at[p].get(mode="promise_in_bounds")

    def butterfly(a):
        s = [a[j] + rgather(a[j], perm1) for j in range(16)]
        c = [jnp.where(m3, s[2 * i], s[2 * i + 1]) for i in range(8)]
        t = [c[j] + rgather(c[j], perm2) for j in range(8)]
        d = [jnp.where(m2, t[2 * i], t[2 * i + 1]) for i in range(4)]
        u = [d[j] + rgather(d[j], perm3) for j in range(4)]
        e = [jnp.where(m1, u[2 * i], u[2 * i + 1]) for i in range(2)]
        v = [e[j] + rgather(e[j], perm4) for j in range(2)]
        r = jnp.where(m0, v[0], v[1])
        return rgather(r, brev)

    def issue(g, par):
        pltpu.async_copy(kv_hbm.at[adj_v.at[g]], rows_v.at[par],
                         sems.at[par])

    def wait(g, par):
        pltpu.make_async_copy(kv_hbm.at[adj_v.at[g]], rows_v.at[par],
                              sems.at[par]).wait()

    def compute_chunk(g, par):
      def qbody(qi, _):
          qrow = g * CH + qi
          roff = qi * D
          qv = [[q_v[qrow, pl.ds(b * DH + j * 16, 16)] for j in range(4)]
                for b in range(2)]

          def unpk(r, woff, j):
              x = plsc.bitcast(rows_v[par, roff + r, pl.ds(woff + j * 16, 16)],
                               jnp.bfloat16)
              return plsc.unpack(x, format=plsc.PackFormat.INTERLEAVED,
                                 preferred_element_type=jnp.float32)

          ws = [[], []]
          for b in range(2):
              koff = b * DH
              masked = []
              for grp in range(4):
                  accs = []
                  for u in range(16):
                      r = grp * 16 + u
                      ev0, od0 = unpk(r, koff, 0)
                      a = ev0 * qv[b][0] + od0 * qv[b][1]
                      ev1, od1 = unpk(r, koff, 1)
                      a = a + ev1 * qv[b][2] + od1 * qv[b][3]
                      accs.append(a)
                  sc = butterfly(accs) * (1.0 / math.sqrt(DH))
                  masked.append(sc + comb_v[qrow, pl.ds(grp * 16, 16)])
              m = jnp.maximum(jnp.maximum(masked[0], masked[1]),
                              jnp.maximum(masked[2], masked[3]))
              mx = jnp.max(m)
              es = [jnp.where(mm > -1e29, jnp.exp(mm - mx),
                              jnp.zeros((16,), jnp.float32))
                    for mm in masked]
              dn = jnp.sum(es[0] + es[1] + es[2] + es[3])
              dnv = jnp.maximum(jnp.full((16,), dn, jnp.float32), 1e-9)
              inv = jnp.full((16,), 1.0, jnp.float32) / dnv
              ws[b] = [e * inv for e in es]

          ys = [jnp.zeros((16,), jnp.float32) for _ in range(8)]
          for d4 in range(4):
              for u in range(16):
                  r = d4 * 16 + u
                  w0 = rgather(ws[0][d4], splats[u])
                  w1 = rgather(ws[1][d4], splats[u])
                  for b, w in ((0, w0), (1, w1)):
                      voff = 32 + b * DH
                      for j in range(2):
                          ev, od = unpk(r, voff, j)
                          ys[b * 4 + 2 * j] = ys[b * 4 + 2 * j] + ev * w
                          ys[b * 4 + 2 * j + 1] = ys[b * 4 + 2 * j + 1] + od * w
          for mreg in range(4):
              out_v[qrow, pl.ds(mreg * 16, 16)] = ys[mreg]
              out_v[qrow, pl.ds(DH + mreg * 16, 16)] = ys[4 + mreg]
          return 0

      lax.fori_loop(0, CH, qbody, 0)

    def sc_body(s, _):
        base = row0 + s * SQ
        t_base = t0 + s * SQ
        pltpu.sync_copy(qf_hbm.at[pl.ds(base, SQ)], q_v)
        pltpu.sync_copy(idx_hbm.at[pl.ds(base, SQ)], idx_v)
        pltpu.sync_copy(et_hbm.at[pl.ds(base, SQ)], et_v)

        def pre_body(i, _):
            for m in range(4):
                qrow = i
                coff = m * 16
                raw = idx_v[qrow, pl.ds(coff, 16)]
                et16 = et_v[qrow, pl.ds(coff, 16)]
                b16 = plsc.load_gather(bias_v, [et16])
                msk = raw <= (t_base + qrow)
                comb_v[qrow, pl.ds(coff, 16)] = jnp.where(
                    msk, b16, jnp.full((16,), NEG, jnp.float32))
                adj_v[i // CH, pl.ds((i % CH) * D + coff, 16)] = raw + hbase
            return 0

        lax.fori_loop(0, SQ, pre_body, 0)

        issue(0, 0)
        issue(1, 1)
        issue(2, 2)

        def ch_body(g, _):
            par = lax.rem(g, 3)
            wait(g, par)
            compute_chunk(g, par)

            @pl.when(g < NCH - 3)
            def _():
                issue(g + 3, par)

            return 0

        lax.fori_loop(0, NCH, ch_body, 0)
        pltpu.sync_copy(out_v, out_hbm.at[pl.ds(base, SQ)])
        return 0

    lax.fori_loop(0, NSC, sc_body, 0)


@functools.partial(
    pl.kernel,
    out_type=jax.ShapeDtypeStruct((H * T, 2 * DH), jnp.float32),
    mesh=plsc.VectorSubcoreMesh(core_axis_name="c", subcore_axis_name="s"),
    compiler_params=pltpu.CompilerParams(needs_layout_passes=False),
    scratch_types=[
        pltpu.VMEM((16,), jnp.float32),
        pltpu.VMEM((SQ, 2 * DH), jnp.float32),
        pltpu.VMEM((SQ, D), jnp.int32),
        pltpu.VMEM((SQ, D), jnp.int32),
        pltpu.VMEM((SQ, D), jnp.float32),
        pltpu.VMEM((NCH, CH * D), jnp.int32),
        pltpu.VMEM((SQ, 2 * DH), jnp.float32),
        pltpu.VMEM((3, CH * D, 2 * DH), jnp.int32),
        pltpu.SemaphoreType.DMA((3,)),
    ],
)
def _sc_attention(kv_hbm, qf_hbm, idx_hbm, et_hbm, bt_hbm, out_hbm,
                  *scratch):
    _attn_kernel(kv_hbm, qf_hbm, idx_hbm, et_hbm, bt_hbm, out_hbm, *scratch)


@jax.jit
def kernel(q, k, v, neigh_idx, edge_type, edge_type_bias):
    kvf = jnp.concatenate([k[0], v[0], k[1], v[1]], axis=-1)
    kvf = kvf.reshape(H * T, 4 * DH).astype(jnp.bfloat16)
    kvp = jax.lax.bitcast_convert_type(
        kvf.reshape(H * T, 2 * DH, 2), jnp.int32)
    qf = jnp.concatenate([q[0], q[1]], axis=-1).reshape(H * T, 2 * DH)
    qd = qf.reshape(H * T, 2, 2, 16, 2).transpose(0, 1, 2, 4, 3)
    qd = qd.reshape(H * T, 2 * DH).astype(jnp.float32)
    idx32 = neigh_idx.astype(jnp.int32).reshape(H * T, D)
    et32 = edge_type.astype(jnp.int32).reshape(H * T, D)
    btab = jnp.zeros((16,), jnp.float32)
    btab = btab.at[1:5].set(edge_type_bias.astype(jnp.float32))
    out = _sc_attention(kvp, qd, idx32, et32, btab)
    y = out.reshape(H, T, 2, 2, 2, 16).transpose(2, 0, 1, 3, 5, 4)
    y = y.reshape(2, H, T, DH)
    return y.astype(v.dtype)
